# Initial kernel scaffold; baseline (speedup 1.0000x reference)
import jax
import jax.numpy as jnp
from jax.experimental import pallas as pl


def kernel(x, edge_index, edge_attr, batch, W0, b0, W1, b1, Wnn, bnn, Wroot, bconv, gru_Wih, gru_Whh, gru_bih, gru_bhh, Wf):
    raise NotImplementedError("write your pallas kernel here")


# speedup vs baseline: 4.1782x; 4.1782x over previous
import functools

import jax
import jax.numpy as jnp
from jax import lax
from jax.experimental import pallas as pl
from jax.experimental.pallas import tpu as pltpu
from jax.experimental.pallas import tpu_sc as plsc

_N = 10000
_E = 320000
_F_IN = 128
_DIM = 64
_DNN = 16
_NG = 64

_NC = 2
_NS = 16
_NW = _NC * _NS
_CH = 128
_NCHUNK = _E // _CH
_RPT = _N // _NS
_RQ = 624

_F32 = jnp.float32


def _sc_conv_body(z_hbm, y_hbm, src_hbm, dst_hbm, attrt_hbm, out_hbm,
                  srcb, dstb, ab, gb, msgb, s_sh, sem):
    cid = lax.axis_index("c")
    sid = lax.axis_index("s")
    wid = sid * _NC + cid

    pltpu.sync_copy(z_hbm.at[pl.ds(sid * _RQ, _RQ)],
                    s_sh.at[pl.ds(sid * _RQ, _RQ)])

    @pl.when(sid == _NS - 1)
    def _ztail():
        pltpu.sync_copy(z_hbm.at[pl.ds(_NS * _RQ, 16)],
                        s_sh.at[pl.ds(_NS * _RQ, 16)])

    def _minit(i, _):
        for jj in range(1, 8):
            msgb[i, pl.ds(jj * 16, 16)] = jnp.zeros((16,), _F32)
        return 0
    lax.fori_loop(0, _CH, _minit, 0)
    plsc.subcore_barrier()

    nchunks = (_NCHUNK - wid + _NW - 1) // _NW

    def _chunk(ci, _):
        chunk = wid + ci * _NW
        pltpu.sync_copy(src_hbm.at[chunk], srcb)
        pltpu.sync_copy(dst_hbm.at[chunk], dstb)
        for k in range(4):
            pltpu.sync_copy(attrt_hbm.at[k, pl.ds(chunk * _CH, _CH)], ab.at[k])
        pltpu.async_copy(y_hbm.at[srcb.at[0]], gb, sem).wait()

        def _group(g, _):
            base_e = g * 16
            a0v = ab[0, pl.ds(base_e, 16)]
            a1v = ab[1, pl.ds(base_e, 16)]
            a2v = ab[2, pl.ds(base_e, 16)]
            a3v = ab[3, pl.ds(base_e, 16)]
            for l in range(16):
                c = base_e + l
                m = (a0v[l] * gb[c, 0:16] + a1v[l] * gb[c, 16:32]
                     + a2v[l] * gb[c, 32:48] + a3v[l] * gb[c, 48:64]
                     + gb[c, 64:80])
                msgb[c, 0:16] = m
            return 0
        lax.fori_loop(0, _CH // 16, _group, 0)

        pltpu.sync_copy(msgb, s_sh.at[dstb.at[0]], add=True)
        return 0

    lax.fori_loop(0, nchunks, _chunk, 0)
    plsc.subcore_barrier()
    pltpu.sync_copy(s_sh.at[pl.ds(sid * _RQ, _RQ)],
                    out_hbm.at[cid, pl.ds(sid * _RQ, _RQ)])

    @pl.when(sid == _NS - 1)
    def _wtail():
        pltpu.sync_copy(s_sh.at[pl.ds(_NS * _RQ, 16)],
                        out_hbm.at[cid, pl.ds(_NS * _RQ, 16)])


def _sc_cnt_body(z_hbm, dst_hbm, out_hbm, dstb, oneb, s_sh):
    cid = lax.axis_index("c")
    sid = lax.axis_index("s")
    wid = sid * _NC + cid

    pltpu.sync_copy(z_hbm.at[pl.ds(sid * _RQ, _RQ)],
                    s_sh.at[pl.ds(sid * _RQ, _RQ)])

    @pl.when(sid == _NS - 1)
    def _ztail():
        pltpu.sync_copy(z_hbm.at[pl.ds(_NS * _RQ, 16)],
                        s_sh.at[pl.ds(_NS * _RQ, 16)])

    def _orow(i, _):
        for jj in range(8):
            oneb[i, pl.ds(jj * 16, 16)] = jnp.ones((16,), _F32)
        return 0
    lax.fori_loop(0, 128, _orow, 0)
    plsc.subcore_barrier()

    nchunks = (_NCHUNK - wid + _NW - 1) // _NW

    def _chunk(ci, _):
        chunk = wid + ci * _NW
        pltpu.sync_copy(dst_hbm.at[chunk], dstb)
        pltpu.sync_copy(oneb, s_sh.at[dstb.at[0]], add=True)
        return 0

    lax.fori_loop(0, nchunks, _chunk, 0)
    plsc.subcore_barrier()
    pltpu.sync_copy(s_sh.at[pl.ds(sid * _RQ, _RQ)],
                    out_hbm.at[cid, pl.ds(sid * _RQ, _RQ)])

    @pl.when(sid == _NS - 1)
    def _wtail():
        pltpu.sync_copy(s_sh.at[pl.ds(_NS * _RQ, 16)],
                        out_hbm.at[cid, pl.ds(_NS * _RQ, 16)])


_SC_MESH = plsc.VectorSubcoreMesh(core_axis_name="c", subcore_axis_name="s")

_sc_conv = pl.kernel(
    _sc_conv_body,
    out_type=jax.ShapeDtypeStruct((_NC, _N, 128), _F32),
    mesh=_SC_MESH,
    scratch_types=[
        pltpu.VMEM((1, 128), jnp.int32),
        pltpu.VMEM((1, 128), jnp.int32),
        pltpu.VMEM((4, _CH), _F32),
        pltpu.VMEM((_CH, 128), _F32),
        pltpu.VMEM((_CH, 128), _F32),
        pltpu.VMEM_SHARED((_N, 128), _F32),
        pltpu.SemaphoreType.DMA,
    ],
)

_sc_cnt = pl.kernel(
    _sc_cnt_body,
    out_type=jax.ShapeDtypeStruct((_NC, _N, 128), _F32),
    mesh=_SC_MESH,
    scratch_types=[
        pltpu.VMEM((1, 128), jnp.int32),
        pltpu.VMEM((128, 128), _F32),
        pltpu.VMEM_SHARED((_N, 128), _F32),
    ],
)


_BR = 1000
_GRID = _N // _BR


def _gru(out2, h, wih, whh, bih, bhh):
    gi = jnp.dot(out2, wih, preferred_element_type=_F32) + bih
    gh = jnp.dot(h, whh, preferred_element_type=_F32) + bhh
    r = jax.nn.sigmoid(gi[:, :_DIM] + gh[:, :_DIM])
    z = jax.nn.sigmoid(gi[:, _DIM:2 * _DIM] + gh[:, _DIM:2 * _DIM])
    n = jnp.tanh(gi[:, 2 * _DIM:] + r * gh[:, 2 * _DIM:])
    return (1.0 - z) * n + z * h


def _tc_init_body(x_ref, w0_ref, b0_ref, wy_ref, by_ref, wr_ref, br_ref,
                  h_ref, y_ref, r_ref):
    out = jnp.maximum(
        jnp.dot(x_ref[...], w0_ref[...], preferred_element_type=_F32)
        + b0_ref[...], 0.0)
    h_ref[...] = out
    y_ref[...] = jnp.dot(out, wy_ref[...], preferred_element_type=_F32) + by_ref[...]
    r_ref[...] = jnp.dot(out, wr_ref[...], preferred_element_type=_F32) + br_ref[...]


def _combine(s_ref, cnt_ref, r_ref):
    s = s_ref[0, :, 0:16] + s_ref[1, :, 0:16]
    cnt = cnt_ref[0, :, 0:1] + cnt_ref[1, :, 0:1]
    return s / jnp.maximum(cnt, 1.0) + r_ref[...]


def _tc_iter_body(s_ref, cnt_ref, r_ref, h_ref, wih_ref, whh_ref, bih_ref,
                  bhh_ref, wy_ref, by_ref, wr_ref, br_ref,
                  h_out, y_out, r_out):
    out2 = _combine(s_ref, cnt_ref, r_ref)
    hnew = _gru(out2, h_ref[...], wih_ref[...], whh_ref[...],
                bih_ref[...], bhh_ref[...])
    h_out[...] = hnew
    y_out[...] = jnp.dot(hnew, wy_ref[...], preferred_element_type=_F32) + by_ref[...]
    r_out[...] = jnp.dot(hnew, wr_ref[...], preferred_element_type=_F32) + br_ref[...]


def _tc_final_body(s_ref, cnt_ref, r_ref, h_ref, wih_ref, whh_ref, bih_ref,
                   bhh_ref, wf_ref, batch_ref, out_ref):
    out2 = _combine(s_ref, cnt_ref, r_ref)
    hnew = _gru(out2, h_ref[...], wih_ref[...], whh_ref[...],
                bih_ref[...], bhh_ref[...])
    res = jnp.dot(hnew, wf_ref[...], preferred_element_type=_F32)
    g = lax.broadcasted_iota(jnp.int32, (_BR, _NG), 1)
    onehot = (batch_ref[...] == g).astype(_F32)
    contrib = jnp.sum(res * onehot, axis=0)[None, :]

    @pl.when(pl.program_id(0) == 0)
    def _():
        out_ref[...] = jnp.zeros_like(out_ref)
    out_ref[...] += contrib


def _row_spec(w):
    return pl.BlockSpec((_BR, w), lambda i: (i, 0))


def _full_spec(shape):
    return pl.BlockSpec(shape, lambda i: tuple(0 for _ in shape))


_S_SPEC = pl.BlockSpec((_NC, _BR, 128), lambda i: (0, i, 0))

_tc_init = pl.pallas_call(
    _tc_init_body,
    grid=(_GRID,),
    in_specs=[_row_spec(_F_IN), _full_spec((_F_IN, _DIM)), _full_spec((1, _DIM)),
              _full_spec((_DIM, 128)), _full_spec((1, 128)),
              _full_spec((_DIM, _DNN)), _full_spec((1, _DNN))],
    out_specs=[_row_spec(_DIM), _row_spec(128), _row_spec(_DNN)],
    out_shape=[jax.ShapeDtypeStruct((_N, _DIM), _F32),
               jax.ShapeDtypeStruct((_N, 128), _F32),
               jax.ShapeDtypeStruct((_N, _DNN), _F32)],
)

_tc_iter = pl.pallas_call(
    _tc_iter_body,
    grid=(_GRID,),
    in_specs=[_S_SPEC, _S_SPEC, _row_spec(_DNN), _row_spec(_DIM),
              _full_spec((_DNN, 3 * _DIM)), _full_spec((_DIM, 3 * _DIM)),
              _full_spec((1, 3 * _DIM)), _full_spec((1, 3 * _DIM)),
              _full_spec((_DIM, 128)), _full_spec((1, 128)),
              _full_spec((_DIM, _DNN)), _full_spec((1, _DNN))],
    out_specs=[_row_spec(_DIM), _row_spec(128), _row_spec(_DNN)],
    out_shape=[jax.ShapeDtypeStruct((_N, _DIM), _F32),
               jax.ShapeDtypeStruct((_N, 128), _F32),
               jax.ShapeDtypeStruct((_N, _DNN), _F32)],
)

_tc_final = pl.pallas_call(
    _tc_final_body,
    grid=(_GRID,),
    in_specs=[_S_SPEC, _S_SPEC, _row_spec(_DNN), _row_spec(_DIM),
              _full_spec((_DNN, 3 * _DIM)), _full_spec((_DIM, 3 * _DIM)),
              _full_spec((1, 3 * _DIM)), _full_spec((1, 3 * _DIM)),
              _full_spec((_DIM, 1)), _row_spec(1)],
    out_specs=pl.BlockSpec((1, _NG), lambda i: (0, 0)),
    out_shape=jax.ShapeDtypeStruct((1, _NG), _F32),
)


def kernel(x, edge_index, edge_attr, batch, W0, b0, W1, b1, Wnn, bnn, Wroot,
           bconv, gru_Wih, gru_Whh, gru_bih, gru_bhh, Wf):
    NL1, NL2 = 4, 2

    fstack = jnp.concatenate(
        [Wnn[k].reshape(_DNN, _DNN) for k in range(4)]
        + [bnn.reshape(_DNN, _DNN)], axis=1)
    wy = jnp.pad(W1 @ fstack, ((0, 0), (0, 48)))
    by = jnp.pad(b1 @ fstack, (0, 48))[None, :]
    wr = W1 @ Wroot
    br = (b1 @ Wroot + bconv)[None, :]
    b0r = b0[None, :]

    src3d = edge_index[0].reshape(_NCHUNK, 1, 128)
    dst3d = edge_index[1].reshape(_NCHUNK, 1, 128)
    attrt = edge_attr.T
    batch2d = batch.reshape(_N, 1)

    zeros_n = jnp.zeros((_N, 128), _F32)
    cnt2 = _sc_cnt(zeros_n, dst3d)

    h, y, r = _tc_init(x, W0, b0r, wy, by, wr, br)

    for j in range(NL1):
        wih = gru_Wih[j].T
        whh = gru_Whh[j].T
        bih = gru_bih[j][None, :]
        bhh = gru_bhh[j][None, :]
        for t in range(NL2):
            s2 = _sc_conv(zeros_n, y, src3d, dst3d, attrt)
            if j == NL1 - 1 and t == NL2 - 1:
                pooled = _tc_final(s2, cnt2, r, h, wih, whh, bih, bhh,
                                   Wf, batch2d)
            else:
                h, y, r = _tc_iter(s2, cnt2, r, h, wih, whh, bih, bhh,
                                   wy, by, wr, br)
    return pooled.reshape(-1)


# speedup vs baseline: 11.1528x; 2.6693x over previous
import functools

import jax
import jax.numpy as jnp
from jax import lax
from jax.experimental import pallas as pl
from jax.experimental.pallas import tpu as pltpu
from jax.experimental.pallas import tpu_sc as plsc

_N = 10000
_E = 320000
_F_IN = 128
_DIM = 64
_DNN = 16
_NG = 64

_NC = 2
_NS = 16
_NW = _NC * _NS
_CH = 128
_NCHUNK = _E // _CH
_RQ = 624

_F32 = jnp.float32


def _sc_conv_body(z_hbm, y_hbm, src_hbm, dst_hbm, attrt_hbm, out_hbm,
                  sb0, sb1, db0, db1, ab0, ab1, gb0, gb1, mb,
                  s_sh, sem_g, sem_s, sem_m):
    cid = lax.axis_index("c")
    sid = lax.axis_index("s")
    wid = sid * _NC + cid

    pltpu.sync_copy(z_hbm.at[pl.ds(sid * _RQ, _RQ)],
                    s_sh.at[pl.ds(sid * _RQ, _RQ)])

    @pl.when(sid == _NS - 1)
    def _ztail():
        pltpu.sync_copy(z_hbm.at[pl.ds(_NS * _RQ, 16)],
                        s_sh.at[pl.ds(_NS * _RQ, 16)])

    def _minit(i, _):
        mb[i, pl.ds(16, 16)] = jnp.ones((16,), _F32)
        for jj in range(2, 8):
            mb[i, pl.ds(jj * 16, 16)] = jnp.zeros((16,), _F32)
        return 0
    lax.fori_loop(0, _CH, _minit, 0)
    plsc.subcore_barrier()

    nchunks = (_NCHUNK - wid + _NW - 1) // _NW

    def _meta_start(ci, sb, db, ab):
        chunk = wid + ci * _NW
        pltpu.async_copy(src_hbm.at[chunk], sb, sem_m)
        pltpu.async_copy(dst_hbm.at[chunk], db, sem_m)
        for k in range(4):
            pltpu.async_copy(attrt_hbm.at[k, pl.ds(chunk * _CH, _CH)],
                             ab.at[k], sem_m)

    def _meta_wait(sb, db, ab):
        pltpu.make_async_copy(src_hbm.at[0], sb, sem_m).wait()
        pltpu.make_async_copy(dst_hbm.at[0], db, sem_m).wait()
        for k in range(4):
            pltpu.make_async_copy(attrt_hbm.at[k, pl.ds(0, _CH)],
                                  ab.at[k], sem_m).wait()

    def _do(ci, sbc, dbc, abc, gbc, sbn, dbn, abn, gbn):
        nxt = ci + 1

        @pl.when(ci >= 1)
        def _():
            pltpu.make_async_copy(mb, s_sh.at[dbn.at[0]], sem_s).wait()

        @pl.when(nxt < nchunks)
        def _():
            _meta_start(nxt, sbn, dbn, abn)

        pltpu.make_async_copy(y_hbm.at[sbc.at[0]], gbc, sem_g).wait()

        @pl.when(nxt < nchunks)
        def _():
            _meta_wait(sbn, dbn, abn)
            pltpu.async_copy(y_hbm.at[sbn.at[0]], gbn, sem_g)

        def _group(g, _):
            base_e = g * 16
            a0v = abc[0, pl.ds(base_e, 16)]
            a1v = abc[1, pl.ds(base_e, 16)]
            a2v = abc[2, pl.ds(base_e, 16)]
            a3v = abc[3, pl.ds(base_e, 16)]
            for l in range(16):
                c = base_e + l
                m = (a0v[l] * gbc[c, 0:16] + a1v[l] * gbc[c, 16:32]
                     + a2v[l] * gbc[c, 32:48] + a3v[l] * gbc[c, 48:64]
                     + gbc[c, 64:80])
                mb[c, 0:16] = m
            return 0
        lax.fori_loop(0, _CH // 16, _group, 0)

        pltpu.async_copy(mb, s_sh.at[dbc.at[0]], sem_s, add=True)

    pltpu.sync_copy(src_hbm.at[wid], sb0)
    pltpu.sync_copy(dst_hbm.at[wid], db0)
    for k in range(4):
        pltpu.sync_copy(attrt_hbm.at[k, pl.ds(wid * _CH, _CH)], ab0.at[k])
    pltpu.async_copy(y_hbm.at[sb0.at[0]], gb0, sem_g)

    def _step(p, _):
        ci = 2 * p
        _do(ci, sb0, db0, ab0, gb0, sb1, db1, ab1, gb1)
        _do(ci + 1, sb1, db1, ab1, gb1, sb0, db0, ab0, gb0)
        return 0
    lax.fori_loop(0, nchunks // 2, _step, 0)

    @pl.when(nchunks % 2 == 1)
    def _last():
        _do(nchunks - 1, sb0, db0, ab0, gb0, sb1, db1, ab1, gb1)

    @pl.when(nchunks % 2 == 1)
    def _dr0():
        pltpu.make_async_copy(mb, s_sh.at[db0.at[0]], sem_s).wait()

    @pl.when(nchunks % 2 == 0)
    def _dr1():
        pltpu.make_async_copy(mb, s_sh.at[db1.at[0]], sem_s).wait()

    plsc.subcore_barrier()
    pltpu.sync_copy(s_sh.at[pl.ds(sid * _RQ, _RQ)],
                    out_hbm.at[cid, pl.ds(sid * _RQ, _RQ)])

    @pl.when(sid == _NS - 1)
    def _wtail():
        pltpu.sync_copy(s_sh.at[pl.ds(_NS * _RQ, 16)],
                        out_hbm.at[cid, pl.ds(_NS * _RQ, 16)])


_SC_MESH = plsc.VectorSubcoreMesh(core_axis_name="c", subcore_axis_name="s")

_sc_conv = pl.kernel(
    _sc_conv_body,
    out_type=jax.ShapeDtypeStruct((_NC, _N, 128), _F32),
    mesh=_SC_MESH,
    scratch_types=[
        pltpu.VMEM((1, _CH), jnp.int32),
        pltpu.VMEM((1, _CH), jnp.int32),
        pltpu.VMEM((1, _CH), jnp.int32),
        pltpu.VMEM((1, _CH), jnp.int32),
        pltpu.VMEM((4, _CH), _F32),
        pltpu.VMEM((4, _CH), _F32),
        pltpu.VMEM((_CH, 128), _F32),
        pltpu.VMEM((_CH, 128), _F32),
        pltpu.VMEM((_CH, 128), _F32),
        pltpu.VMEM_SHARED((_N, 128), _F32),
        pltpu.SemaphoreType.DMA,
        pltpu.SemaphoreType.DMA,
        pltpu.SemaphoreType.DMA,
    ],
)


_BR = 1000
_GRID = _N // _BR


def _gru(out2, h, wih, whh, bih, bhh):
    gi = jnp.dot(out2, wih, preferred_element_type=_F32) + bih
    gh = jnp.dot(h, whh, preferred_element_type=_F32) + bhh
    r = jax.nn.sigmoid(gi[:, :_DIM] + gh[:, :_DIM])
    z = jax.nn.sigmoid(gi[:, _DIM:2 * _DIM] + gh[:, _DIM:2 * _DIM])
    n = jnp.tanh(gi[:, 2 * _DIM:] + r * gh[:, 2 * _DIM:])
    return (1.0 - z) * n + z * h


def _tc_init_body(x_ref, w0_ref, b0_ref, wy_ref, by_ref, wr_ref, br_ref,
                  h_ref, y_ref, r_ref):
    out = jnp.maximum(
        jnp.dot(x_ref[...], w0_ref[...], preferred_element_type=_F32)
        + b0_ref[...], 0.0)
    h_ref[...] = out
    y_ref[...] = jnp.dot(out, wy_ref[...], preferred_element_type=_F32) + by_ref[...]
    r_ref[...] = jnp.dot(out, wr_ref[...], preferred_element_type=_F32) + br_ref[...]


def _combine(s_ref, r_ref):
    s = s_ref[0, :, 0:16] + s_ref[1, :, 0:16]
    cnt = s_ref[0, :, 16:17] + s_ref[1, :, 16:17]
    return s / jnp.maximum(cnt, 1.0) + r_ref[...]


def _tc_iter_body(s_ref, r_ref, h_ref, wih_ref, whh_ref, bih_ref,
                  bhh_ref, wy_ref, by_ref, wr_ref, br_ref,
                  h_out, y_out, r_out):
    out2 = _combine(s_ref, r_ref)
    hnew = _gru(out2, h_ref[...], wih_ref[...], whh_ref[...],
                bih_ref[...], bhh_ref[...])
    h_out[...] = hnew
    y_out[...] = jnp.dot(hnew, wy_ref[...], preferred_element_type=_F32) + by_ref[...]
    r_out[...] = jnp.dot(hnew, wr_ref[...], preferred_element_type=_F32) + br_ref[...]


def _tc_final_body(s_ref, r_ref, h_ref, wih_ref, whh_ref, bih_ref,
                   bhh_ref, wf_ref, batch_ref, out_ref):
    out2 = _combine(s_ref, r_ref)
    hnew = _gru(out2, h_ref[...], wih_ref[...], whh_ref[...],
                bih_ref[...], bhh_ref[...])
    res = jnp.dot(hnew, wf_ref[...], preferred_element_type=_F32)
    g = lax.broadcasted_iota(jnp.int32, (_BR, _NG), 1)
    onehot = (batch_ref[...] == g).astype(_F32)
    contrib = jnp.sum(res * onehot, axis=0)[None, :]

    @pl.when(pl.program_id(0) == 0)
    def _():
        out_ref[...] = jnp.zeros_like(out_ref)
    out_ref[...] += contrib


def _row_spec(w):
    return pl.BlockSpec((_BR, w), lambda i: (i, 0))


def _full_spec(shape):
    return pl.BlockSpec(shape, lambda i: tuple(0 for _ in shape))


_S_SPEC = pl.BlockSpec((_NC, _BR, 128), lambda i: (0, i, 0))

_tc_init = pl.pallas_call(
    _tc_init_body,
    grid=(_GRID,),
    in_specs=[_row_spec(_F_IN), _full_spec((_F_IN, _DIM)), _full_spec((1, _DIM)),
              _full_spec((_DIM, 128)), _full_spec((1, 128)),
              _full_spec((_DIM, _DNN)), _full_spec((1, _DNN))],
    out_specs=[_row_spec(_DIM), _row_spec(128), _row_spec(_DNN)],
    out_shape=[jax.ShapeDtypeStruct((_N, _DIM), _F32),
               jax.ShapeDtypeStruct((_N, 128), _F32),
               jax.ShapeDtypeStruct((_N, _DNN), _F32)],
)

_tc_iter = pl.pallas_call(
    _tc_iter_body,
    grid=(_GRID,),
    in_specs=[_S_SPEC, _row_spec(_DNN), _row_spec(_DIM),
              _full_spec((_DNN, 3 * _DIM)), _full_spec((_DIM, 3 * _DIM)),
              _full_spec((1, 3 * _DIM)), _full_spec((1, 3 * _DIM)),
              _full_spec((_DIM, 128)), _full_spec((1, 128)),
              _full_spec((_DIM, _DNN)), _full_spec((1, _DNN))],
    out_specs=[_row_spec(_DIM), _row_spec(128), _row_spec(_DNN)],
    out_shape=[jax.ShapeDtypeStruct((_N, _DIM), _F32),
               jax.ShapeDtypeStruct((_N, 128), _F32),
               jax.ShapeDtypeStruct((_N, _DNN), _F32)],
)

_tc_final = pl.pallas_call(
    _tc_final_body,
    grid=(_GRID,),
    in_specs=[_S_SPEC, _row_spec(_DNN), _row_spec(_DIM),
              _full_spec((_DNN, 3 * _DIM)), _full_spec((_DIM, 3 * _DIM)),
              _full_spec((1, 3 * _DIM)), _full_spec((1, 3 * _DIM)),
              _full_spec((_DIM, 1)), _row_spec(1)],
    out_specs=pl.BlockSpec((1, _NG), lambda i: (0, 0)),
    out_shape=jax.ShapeDtypeStruct((1, _NG), _F32),
)


def kernel(x, edge_index, edge_attr, batch, W0, b0, W1, b1, Wnn, bnn, Wroot,
           bconv, gru_Wih, gru_Whh, gru_bih, gru_bhh, Wf):
    NL1, NL2 = 4, 2

    fstack = jnp.concatenate(
        [Wnn[k].reshape(_DNN, _DNN) for k in range(4)]
        + [bnn.reshape(_DNN, _DNN)], axis=1)
    wy = jnp.pad(W1 @ fstack, ((0, 0), (0, 48)))
    by = jnp.pad(b1 @ fstack, (0, 48))[None, :]
    wr = W1 @ Wroot
    br = (b1 @ Wroot + bconv)[None, :]
    b0r = b0[None, :]

    src3d = edge_index[0].reshape(_NCHUNK, 1, _CH)
    dst3d = edge_index[1].reshape(_NCHUNK, 1, _CH)
    attrt = edge_attr.T
    batch2d = batch.reshape(_N, 1)

    zeros_n = jnp.zeros((_N, 128), _F32)

    h, y, r = _tc_init(x, W0, b0r, wy, by, wr, br)

    for j in range(NL1):
        wih = gru_Wih[j].T
        whh = gru_Whh[j].T
        bih = gru_bih[j][None, :]
        bhh = gru_bhh[j][None, :]
        for t in range(NL2):
            s2 = _sc_conv(zeros_n, y, src3d, dst3d, attrt)
            if j == NL1 - 1 and t == NL2 - 1:
                pooled = _tc_final(s2, r, h, wih, whh, bih, bhh,
                                   Wf, batch2d)
            else:
                h, y, r = _tc_iter(s2, r, h, wih, whh, bih, bhh,
                                   wy, by, wr, br)
    return pooled.reshape(-1)


# speedup vs baseline: 12.4708x; 1.1182x over previous
import functools

import jax
import jax.numpy as jnp
from jax import lax
from jax.experimental import pallas as pl
from jax.experimental.pallas import tpu as pltpu
from jax.experimental.pallas import tpu_sc as plsc

_N = 10000
_E = 320000
_F_IN = 128
_DIM = 64
_DNN = 16
_NG = 64

_NC = 2
_NS = 16
_NW = _NC * _NS
_CH = 128
_NCHUNK = _E // _CH
_RQ = 624

_F32 = jnp.float32


def _sc_conv_body(z_hbm, y_hbm, src_hbm, dst_hbm, attrt_hbm, out_hbm,
                  sb0, sb1, db0, db1, ab0, ab1, gb0, gb1, mb,
                  s_sh, sem_g, sem_s, sem_m, sem_src):
    cid = lax.axis_index("c")
    sid = lax.axis_index("s")
    wid = sid * _NC + cid

    pltpu.sync_copy(z_hbm.at[pl.ds(sid * _RQ, _RQ)],
                    s_sh.at[pl.ds(sid * _RQ, _RQ)])

    @pl.when(sid == _NS - 1)
    def _ztail():
        pltpu.sync_copy(z_hbm.at[pl.ds(_NS * _RQ, 16)],
                        s_sh.at[pl.ds(_NS * _RQ, 16)])

    def _minit(i, _):
        mb[i, pl.ds(16, 16)] = jnp.ones((16,), _F32)
        for jj in range(2, 8):
            mb[i, pl.ds(jj * 16, 16)] = jnp.zeros((16,), _F32)
        return 0
    lax.fori_loop(0, _CH, _minit, 0)
    plsc.subcore_barrier()

    nchunks = (_NCHUNK - wid + _NW - 1) // _NW

    def _do(ci, sbc, dbc, abc, gbc, sbn, dbn, abn, gbn):
        nxt = ci + 1

        @pl.when(ci >= 1)
        def _():
            pltpu.make_async_copy(mb, s_sh.at[dbn.at[0]], sem_s).wait()

        pltpu.make_async_copy(y_hbm.at[sbc.at[0]], gbc, sem_g).wait()

        @pl.when(nxt < nchunks)
        def _():
            pltpu.make_async_copy(src_hbm.at[0], sbn, sem_src).wait()
            pltpu.async_copy(y_hbm.at[sbn.at[0]], gbn, sem_g)

        @pl.when(ci + 2 < nchunks)
        def _():
            pltpu.async_copy(src_hbm.at[wid + (ci + 2) * _NW], sbc, sem_src)

        @pl.when(ci >= 1)
        def _():
            pltpu.make_async_copy(dst_hbm.at[0], dbc, sem_m).wait()
            for k in range(4):
                pltpu.make_async_copy(attrt_hbm.at[k, pl.ds(0, _CH)],
                                      abc.at[k], sem_m).wait()

        @pl.when(nxt < nchunks)
        def _():
            nchunk = wid + nxt * _NW
            pltpu.async_copy(dst_hbm.at[nchunk], dbn, sem_m)
            for k in range(4):
                pltpu.async_copy(attrt_hbm.at[k, pl.ds(nchunk * _CH, _CH)],
                                 abn.at[k], sem_m)

        def _group(g, _):
            base_e = g * 16
            a0v = abc[0, pl.ds(base_e, 16)]
            a1v = abc[1, pl.ds(base_e, 16)]
            a2v = abc[2, pl.ds(base_e, 16)]
            a3v = abc[3, pl.ds(base_e, 16)]
            for l in range(16):
                c = base_e + l
                m = (a0v[l] * gbc[c, 0:16] + a1v[l] * gbc[c, 16:32]
                     + a2v[l] * gbc[c, 32:48] + a3v[l] * gbc[c, 48:64]
                     + gbc[c, 64:80])
                mb[c, 0:16] = m
            return 0
        lax.fori_loop(0, _CH // 16, _group, 0)

        pltpu.async_copy(mb, s_sh.at[dbc.at[0]], sem_s, add=True)

    pltpu.sync_copy(src_hbm.at[wid], sb0)
    pltpu.sync_copy(dst_hbm.at[wid], db0)
    for k in range(4):
        pltpu.sync_copy(attrt_hbm.at[k, pl.ds(wid * _CH, _CH)], ab0.at[k])
    pltpu.async_copy(y_hbm.at[sb0.at[0]], gb0, sem_g)

    @pl.when(nchunks > 1)
    def _pro1():
        pltpu.async_copy(src_hbm.at[wid + _NW], sb1, sem_src)

    def _step(p, _):
        ci = 2 * p
        _do(ci, sb0, db0, ab0, gb0, sb1, db1, ab1, gb1)
        _do(ci + 1, sb1, db1, ab1, gb1, sb0, db0, ab0, gb0)
        return 0
    lax.fori_loop(0, nchunks // 2, _step, 0)

    @pl.when(nchunks % 2 == 1)
    def _last():
        _do(nchunks - 1, sb0, db0, ab0, gb0, sb1, db1, ab1, gb1)

    @pl.when(nchunks % 2 == 1)
    def _dr0():
        pltpu.make_async_copy(mb, s_sh.at[db0.at[0]], sem_s).wait()

    @pl.when(nchunks % 2 == 0)
    def _dr1():
        pltpu.make_async_copy(mb, s_sh.at[db1.at[0]], sem_s).wait()

    plsc.subcore_barrier()
    pltpu.sync_copy(s_sh.at[pl.ds(sid * _RQ, _RQ)],
                    out_hbm.at[cid, pl.ds(sid * _RQ, _RQ)])

    @pl.when(sid == _NS - 1)
    def _wtail():
        pltpu.sync_copy(s_sh.at[pl.ds(_NS * _RQ, 16)],
                        out_hbm.at[cid, pl.ds(_NS * _RQ, 16)])


_SC_MESH = plsc.VectorSubcoreMesh(core_axis_name="c", subcore_axis_name="s")

_sc_conv = pl.kernel(
    _sc_conv_body,
    out_type=jax.ShapeDtypeStruct((_NC, _N, 128), _F32),
    mesh=_SC_MESH,
    scratch_types=[
        pltpu.VMEM((1, _CH), jnp.int32),
        pltpu.VMEM((1, _CH), jnp.int32),
        pltpu.VMEM((1, _CH), jnp.int32),
        pltpu.VMEM((1, _CH), jnp.int32),
        pltpu.VMEM((4, _CH), _F32),
        pltpu.VMEM((4, _CH), _F32),
        pltpu.VMEM((_CH, 128), _F32),
        pltpu.VMEM((_CH, 128), _F32),
        pltpu.VMEM((_CH, 128), _F32),
        pltpu.VMEM_SHARED((_N, 128), _F32),
        pltpu.SemaphoreType.DMA,
        pltpu.SemaphoreType.DMA,
        pltpu.SemaphoreType.DMA,
        pltpu.SemaphoreType.DMA,
    ],
)


_BR = 1000
_GRID = _N // _BR


def _gru(out2, h, wih, whh, bih, bhh):
    gi = jnp.dot(out2, wih, preferred_element_type=_F32) + bih
    gh = jnp.dot(h, whh, preferred_element_type=_F32) + bhh
    r = jax.nn.sigmoid(gi[:, :_DIM] + gh[:, :_DIM])
    z = jax.nn.sigmoid(gi[:, _DIM:2 * _DIM] + gh[:, _DIM:2 * _DIM])
    n = jnp.tanh(gi[:, 2 * _DIM:] + r * gh[:, 2 * _DIM:])
    return (1.0 - z) * n + z * h


def _tc_init_body(x_ref, w0_ref, b0_ref, wy_ref, by_ref, wr_ref, br_ref,
                  h_ref, y_ref, r_ref):
    out = jnp.maximum(
        jnp.dot(x_ref[...], w0_ref[...], preferred_element_type=_F32)
        + b0_ref[...], 0.0)
    h_ref[...] = out
    y_ref[...] = jnp.dot(out, wy_ref[...], preferred_element_type=_F32) + by_ref[...]
    r_ref[...] = jnp.dot(out, wr_ref[...], preferred_element_type=_F32) + br_ref[...]


def _combine(s_ref, r_ref):
    s = s_ref[0, :, 0:16] + s_ref[1, :, 0:16]
    cnt = s_ref[0, :, 16:17] + s_ref[1, :, 16:17]
    return s / jnp.maximum(cnt, 1.0) + r_ref[...]


def _tc_iter_body(s_ref, r_ref, h_ref, wih_ref, whh_ref, bih_ref,
                  bhh_ref, wy_ref, by_ref, wr_ref, br_ref,
                  h_out, y_out, r_out):
    out2 = _combine(s_ref, r_ref)
    hnew = _gru(out2, h_ref[...], wih_ref[...], whh_ref[...],
                bih_ref[...], bhh_ref[...])
    h_out[...] = hnew
    y_out[...] = jnp.dot(hnew, wy_ref[...], preferred_element_type=_F32) + by_ref[...]
    r_out[...] = jnp.dot(hnew, wr_ref[...], preferred_element_type=_F32) + br_ref[...]


def _tc_final_body(s_ref, r_ref, h_ref, wih_ref, whh_ref, bih_ref,
                   bhh_ref, wf_ref, batch_ref, out_ref):
    out2 = _combine(s_ref, r_ref)
    hnew = _gru(out2, h_ref[...], wih_ref[...], whh_ref[...],
                bih_ref[...], bhh_ref[...])
    res = jnp.dot(hnew, wf_ref[...], preferred_element_type=_F32)
    g = lax.broadcasted_iota(jnp.int32, (_BR, _NG), 1)
    onehot = (batch_ref[...] == g).astype(_F32)
    contrib = jnp.sum(res * onehot, axis=0)[None, :]

    @pl.when(pl.program_id(0) == 0)
    def _():
        out_ref[...] = jnp.zeros_like(out_ref)
    out_ref[...] += contrib


def _row_spec(w):
    return pl.BlockSpec((_BR, w), lambda i: (i, 0))


def _full_spec(shape):
    return pl.BlockSpec(shape, lambda i: tuple(0 for _ in shape))


_S_SPEC = pl.BlockSpec((_NC, _BR, 128), lambda i: (0, i, 0))

_tc_init = pl.pallas_call(
    _tc_init_body,
    grid=(_GRID,),
    in_specs=[_row_spec(_F_IN), _full_spec((_F_IN, _DIM)), _full_spec((1, _DIM)),
              _full_spec((_DIM, 128)), _full_spec((1, 128)),
              _full_spec((_DIM, _DNN)), _full_spec((1, _DNN))],
    out_specs=[_row_spec(_DIM), _row_spec(128), _row_spec(_DNN)],
    out_shape=[jax.ShapeDtypeStruct((_N, _DIM), _F32),
               jax.ShapeDtypeStruct((_N, 128), _F32),
               jax.ShapeDtypeStruct((_N, _DNN), _F32)],
)

_tc_iter = pl.pallas_call(
    _tc_iter_body,
    grid=(_GRID,),
    in_specs=[_S_SPEC, _row_spec(_DNN), _row_spec(_DIM),
              _full_spec((_DNN, 3 * _DIM)), _full_spec((_DIM, 3 * _DIM)),
              _full_spec((1, 3 * _DIM)), _full_spec((1, 3 * _DIM)),
              _full_spec((_DIM, 128)), _full_spec((1, 128)),
              _full_spec((_DIM, _DNN)), _full_spec((1, _DNN))],
    out_specs=[_row_spec(_DIM), _row_spec(128), _row_spec(_DNN)],
    out_shape=[jax.ShapeDtypeStruct((_N, _DIM), _F32),
               jax.ShapeDtypeStruct((_N, 128), _F32),
               jax.ShapeDtypeStruct((_N, _DNN), _F32)],
)

_tc_final = pl.pallas_call(
    _tc_final_body,
    grid=(_GRID,),
    in_specs=[_S_SPEC, _row_spec(_DNN), _row_spec(_DIM),
              _full_spec((_DNN, 3 * _DIM)), _full_spec((_DIM, 3 * _DIM)),
              _full_spec((1, 3 * _DIM)), _full_spec((1, 3 * _DIM)),
              _full_spec((_DIM, 1)), _row_spec(1)],
    out_specs=pl.BlockSpec((1, _NG), lambda i: (0, 0)),
    out_shape=jax.ShapeDtypeStruct((1, _NG), _F32),
)


def kernel(x, edge_index, edge_attr, batch, W0, b0, W1, b1, Wnn, bnn, Wroot,
           bconv, gru_Wih, gru_Whh, gru_bih, gru_bhh, Wf):
    NL1, NL2 = 4, 2

    fstack = jnp.concatenate(
        [Wnn[k].reshape(_DNN, _DNN) for k in range(4)]
        + [bnn.reshape(_DNN, _DNN)], axis=1)
    wy = jnp.pad(W1 @ fstack, ((0, 0), (0, 48)))
    by = jnp.pad(b1 @ fstack, (0, 48))[None, :]
    wr = W1 @ Wroot
    br = (b1 @ Wroot + bconv)[None, :]
    b0r = b0[None, :]

    src3d = edge_index[0].reshape(_NCHUNK, 1, _CH)
    dst3d = edge_index[1].reshape(_NCHUNK, 1, _CH)
    attrt = edge_attr.T
    batch2d = batch.reshape(_N, 1)

    zeros_n = jnp.zeros((_N, 128), _F32)

    h, y, r = _tc_init(x, W0, b0r, wy, by, wr, br)

    for j in range(NL1):
        wih = gru_Wih[j].T
        whh = gru_Whh[j].T
        bih = gru_bih[j][None, :]
        bhh = gru_bhh[j][None, :]
        for t in range(NL2):
            s2 = _sc_conv(zeros_n, y, src3d, dst3d, attrt)
            if j == NL1 - 1 and t == NL2 - 1:
                pooled = _tc_final(s2, r, h, wih, whh, bih, bhh,
                                   Wf, batch2d)
            else:
                h, y, r = _tc_iter(s2, r, h, wih, whh, bih, bhh,
                                   wy, by, wr, br)
    return pooled.reshape(-1)
